# baseline (speedup 1.0000x reference)
import functools

import jax
import jax.numpy as jnp
from jax import lax
from jax.experimental import pallas as pl
from jax.experimental.pallas import tpu as pltpu
from jax.experimental.pallas import tpu_sc as plsc

H = 128
N = 10240
E = 163840
ED = 16
G = 40
VOCAB = 256

EDGE_BLK = 2048
N_BLK = 2048


def _layer0_body(rowm_ref, colm_ref, eaT_ref, emb_ref, encW_ref, encb_ref,
                 eW1_ref, eb1_ref, eW2_ref, eb2_ref,
                 nW1_ref, nb1_ref, nW2_ref, nb2_ref,
                 zW_ref, zb_ref,
                 z_ref, msg_ref):
    bf = jnp.bfloat16
    rowm = rowm_ref[0, 0, :]
    colm = colm_ref[0, 0, :]
    ids = lax.broadcasted_iota(jnp.int32, (EDGE_BLK, VOCAB), 1)
    oh_r = (ids == rowm[:, None]).astype(bf)
    oh_c = (ids == colm[:, None]).astype(bf)
    emb = emb_ref[...].astype(bf)
    xr = jnp.dot(oh_r, emb, preferred_element_type=jnp.float32)
    xc = jnp.dot(oh_c, emb, preferred_element_type=jnp.float32)
    e0 = lax.dot_general(eaT_ref[...], encW_ref[...], (((0,), (0,)), ((), ())),
                         preferred_element_type=jnp.float32) + encb_ref[...]
    W1 = eW1_ref[...].astype(bf)
    h = (jnp.dot(xr.astype(bf), W1[0:H], preferred_element_type=jnp.float32)
         + jnp.dot(xc.astype(bf), W1[H:2 * H], preferred_element_type=jnp.float32)
         + jnp.dot(e0.astype(bf), W1[2 * H:3 * H], preferred_element_type=jnp.float32)
         + eb1_ref[...])
    h = jnp.maximum(h, 0.0)
    e1 = jnp.dot(h.astype(bf), eW2_ref[...].astype(bf),
                 preferred_element_type=jnp.float32) + eb2_ref[...]
    nW1 = nW1_ref[...].astype(bf)
    m = (jnp.dot(xr.astype(bf), nW1[0:H], preferred_element_type=jnp.float32)
         + jnp.dot(e1.astype(bf), nW1[H:2 * H], preferred_element_type=jnp.float32)
         + nb1_ref[...])
    m = jnp.maximum(m, 0.0)
    msg = jnp.dot(m.astype(bf), nW2_ref[...].astype(bf),
                  preferred_element_type=jnp.float32) + nb2_ref[...]
    z = jnp.dot(e1, zW_ref[...], preferred_element_type=jnp.float32) + zb_ref[...]
    z_ref[...] = z.astype(jnp.bfloat16)
    msg_ref[...] = msg


def _layer1_body(yr_ref, z_ref, nW2_ref, nb2_ref, msg_ref):
    m = jnp.maximum(yr_ref[...] + z_ref[...].astype(jnp.float32), 0.0)
    msg_ref[...] = jnp.dot(m, nW2_ref[...], preferred_element_type=jnp.float32) + nb2_ref[...]


def _head_body(ga_ref, gb_ref, gc_ref, gd_ref,
               rW1_ref, rb1_ref, rW2_ref, rb2_ref, out_ref):
    g = ga_ref[...] + gb_ref[...] + gc_ref[...] + gd_ref[...]
    hmid = jnp.maximum(jnp.dot(g, rW1_ref[...], preferred_element_type=jnp.float32)
                       + rb1_ref[...], 0.0)
    out_ref[...] = jnp.dot(hmid, rW2_ref[...], preferred_element_type=jnp.float32) + rb2_ref[...]


NW = 32
EPW = E // NW
K = 128
CHUNKS = EPW // K
S_BLKR = 128
S_SUB = S_BLKR // K
S_GROUPS = EPW // S_BLKR
G_BLKR = 256
G_SUB = G_BLKR // K
G_GROUPS = EPW // G_BLKR
NPS = N // 16


def _sc_mesh():
    return plsc.VectorSubcoreMesh(core_axis_name="c", subcore_axis_name="s")


def _scatter_call(msg, col3, zrows):
    epw = msg.shape[0] // NW
    sgroups = epw // S_BLKR

    @functools.partial(
        pl.kernel,
        out_type=jax.ShapeDtypeStruct((2, N, H), jnp.float32),
        mesh=_sc_mesh(),
        scratch_types=[
            pltpu.VMEM((2, S_SUB, K), jnp.int32),
            pltpu.VMEM((2, S_BLKR, H), jnp.float32),
            pltpu.VMEM_SHARED((N, H), jnp.float32),
            pltpu.SemaphoreType.DMA,
            pltpu.SemaphoreType.DMA,
            pltpu.SemaphoreType.DMA,
        ],
    )
    def k(msg_hbm, col_hbm, z_hbm, out_hbm, idx_v, rows_v, shared, sf0, sf1, ss):
        cid = lax.axis_index("c")
        sid = lax.axis_index("s")
        wid = sid * 2 + cid
        base_e = wid * epw
        sf = (sf0, sf1)

        pltpu.async_copy(col_hbm.at[wid, pl.ds(0, S_SUB)], idx_v.at[0], sf0)
        pltpu.async_copy(msg_hbm.at[pl.ds(base_e, S_BLKR)], rows_v.at[0], sf0)
        pltpu.sync_copy(z_hbm, shared.at[pl.ds(sid * NPS, NPS)])
        plsc.subcore_barrier()

        @pl.loop(0, sgroups, step=2)
        def _(g):
            for b in range(2):
                gg = g + b
                nb = 1 - b
                pltpu.make_async_copy(col_hbm.at[wid, pl.ds(0, S_SUB)],
                                      idx_v.at[b], sf[b]).wait()
                pltpu.make_async_copy(msg_hbm.at[pl.ds(base_e, S_BLKR)],
                                      rows_v.at[b], sf[b]).wait()

                @pl.when(gg + 1 < sgroups)
                def _():
                    pltpu.async_copy(col_hbm.at[wid, pl.ds((gg + 1) * S_SUB, S_SUB)],
                                     idx_v.at[nb], sf[nb])
                    pltpu.async_copy(
                        msg_hbm.at[pl.ds(base_e + (gg + 1) * S_BLKR, S_BLKR)],
                        rows_v.at[nb], sf[nb])

                for j in range(S_SUB):
                    pltpu.async_copy(rows_v.at[b, pl.ds(j * K, K)],
                                     shared.at[idx_v.at[b, j]], ss, add=True)
                for j in range(S_SUB):
                    pltpu.make_async_copy(rows_v.at[b, pl.ds(j * K, K)],
                                          shared.at[pl.ds(0, K)], ss).wait()

        plsc.subcore_barrier()
        pltpu.sync_copy(shared.at[pl.ds(sid * NPS, NPS)],
                        out_hbm.at[cid, pl.ds(sid * NPS, NPS)])

    return k(msg, col3, zrows)


def _pool_scatter_call(msg, col3, batch, zrows):
    epw = msg.shape[0] // NW
    sgroups = epw // S_BLKR

    @functools.partial(
        pl.kernel,
        out_type=jax.ShapeDtypeStruct((2, G, H), jnp.float32),
        mesh=_sc_mesh(),
        scratch_types=[
            pltpu.VMEM((2, S_SUB, K), jnp.int32),
            pltpu.VMEM((2, S_SUB, K), jnp.int32),
            pltpu.VMEM((2, S_BLKR, H), jnp.float32),
            pltpu.VMEM_SHARED((G, H), jnp.float32),
            pltpu.SemaphoreType.DMA,
            pltpu.SemaphoreType.DMA,
            pltpu.SemaphoreType.DMA,
            pltpu.SemaphoreType.DMA,
            pltpu.SemaphoreType.DMA,
        ],
    )
    def k(msg_hbm, col_hbm, b_hbm, z_hbm, out_hbm,
          idx_v, bc_v, rows_v, gacc, sf0, sf1, sb0, sb1, ss):
        cid = lax.axis_index("c")
        sid = lax.axis_index("s")
        wid = sid * 2 + cid
        base_e = wid * epw
        sf = (sf0, sf1)
        sb = (sb0, sb1)

        pltpu.async_copy(col_hbm.at[wid, pl.ds(0, S_SUB)], idx_v.at[0], sf0)
        pltpu.async_copy(msg_hbm.at[pl.ds(base_e, S_BLKR)], rows_v.at[0], sf0)

        @pl.when(sid == 0)
        def _():
            pltpu.sync_copy(z_hbm.at[pl.ds(0, G)], gacc)

        plsc.subcore_barrier()

        @pl.loop(0, sgroups, step=2)
        def _(g):
            for b in range(2):
                gg = g + b
                nb = 1 - b
                pltpu.make_async_copy(col_hbm.at[wid, pl.ds(0, S_SUB)],
                                      idx_v.at[b], sf[b]).wait()
                pltpu.make_async_copy(msg_hbm.at[pl.ds(base_e, S_BLKR)],
                                      rows_v.at[b], sf[b]).wait()

                for j in range(S_SUB):
                    pltpu.async_copy(b_hbm.at[idx_v.at[b, j]], bc_v.at[b, j], sb[b])

                @pl.when(gg + 1 < sgroups)
                def _():
                    pltpu.async_copy(col_hbm.at[wid, pl.ds((gg + 1) * S_SUB, S_SUB)],
                                     idx_v.at[nb], sf[nb])
                    pltpu.async_copy(
                        msg_hbm.at[pl.ds(base_e + (gg + 1) * S_BLKR, S_BLKR)],
                        rows_v.at[nb], sf[nb])

                for j in range(S_SUB):
                    pltpu.make_async_copy(b_hbm.at[pl.ds(0, K)],
                                          bc_v.at[b, j], sb[b]).wait()
                for j in range(S_SUB):
                    pltpu.async_copy(rows_v.at[b, pl.ds(j * K, K)],
                                     gacc.at[bc_v.at[b, j]], ss, add=True)
                for j in range(S_SUB):
                    pltpu.make_async_copy(msg_hbm.at[pl.ds(0, K)],
                                          rows_v.at[b, pl.ds(j * K, K)], ss).wait()

        plsc.subcore_barrier()

        @pl.when(sid == 0)
        def _():
            pltpu.sync_copy(gacc, out_hbm.at[cid])

    return k(msg, col3, batch, zrows)


def _gather_call(x1, row3):
    nE = row3.shape[0] * row3.shape[1] * row3.shape[2]
    epw = nE // NW
    ggroups = epw // G_BLKR

    @functools.partial(
        pl.kernel,
        out_type=jax.ShapeDtypeStruct((nE, H), jnp.float32),
        mesh=_sc_mesh(),
        scratch_types=[
            pltpu.VMEM((2, G_SUB, K), jnp.int32),
            pltpu.VMEM((2, G_BLKR, H), jnp.float32),
            pltpu.SemaphoreType.DMA,
            pltpu.SemaphoreType.DMA,
            pltpu.SemaphoreType.DMA,
            pltpu.SemaphoreType.DMA,
            pltpu.SemaphoreType.DMA,
        ],
    )
    def k(x_hbm, row_hbm, out_hbm, idx_v, rows_v, si0, si1, sg, so0, so1):
        cid = lax.axis_index("c")
        sid = lax.axis_index("s")
        wid = sid * 2 + cid
        base_e = wid * epw
        si = (si0, si1)
        so = (so0, so1)

        pltpu.async_copy(row_hbm.at[wid, pl.ds(0, G_SUB)], idx_v.at[0], si0)

        @pl.loop(0, ggroups, step=2)
        def _(g):
            for b in range(2):
                gg = g + b
                nb = 1 - b
                pltpu.make_async_copy(row_hbm.at[wid, pl.ds(0, G_SUB)],
                                      idx_v.at[b], si[b]).wait()

                @pl.when(gg + 1 < ggroups)
                def _():
                    pltpu.async_copy(row_hbm.at[wid, pl.ds((gg + 1) * G_SUB, G_SUB)],
                                     idx_v.at[nb], si[nb])

                @pl.when(gg >= 2)
                def _():
                    pltpu.make_async_copy(rows_v.at[b],
                                          out_hbm.at[pl.ds(0, G_BLKR)], so[b]).wait()

                for j in range(G_SUB):
                    pltpu.async_copy(x_hbm.at[idx_v.at[b, j]],
                                     rows_v.at[b, pl.ds(j * K, K)], sg)
                for j in range(G_SUB):
                    pltpu.make_async_copy(x_hbm.at[pl.ds(0, K)],
                                          rows_v.at[b, pl.ds(j * K, K)], sg).wait()
                pltpu.async_copy(rows_v.at[b],
                                 out_hbm.at[pl.ds(base_e + gg * G_BLKR, G_BLKR)], so[b])

        for b in range(2):
            pltpu.make_async_copy(rows_v.at[b],
                                  out_hbm.at[pl.ds(0, G_BLKR)], so[b]).wait()

    return k(x1, row3)


def _combine_body(a_ref, b_ref, c_ref, d_ref, dW_ref, y_ref):
    y_ref[...] = jnp.dot(a_ref[...] + b_ref[...] + c_ref[...] + d_ref[...],
                         dW_ref[...], preferred_element_type=jnp.float32)


def _combine_call(pa, pb, dW):
    grid = (N // N_BLK,)
    nblk = pl.BlockSpec((N_BLK, H), lambda i: (i, 0))
    return pl.pallas_call(
        _combine_body,
        grid=grid,
        in_specs=[nblk, nblk, nblk, nblk, _full((H, H))],
        out_specs=nblk,
        out_shape=jax.ShapeDtypeStruct((N, H), jnp.float32),
    )(pa[0], pa[1], pb[0], pb[1], dW)


def _full(shape):
    return pl.BlockSpec(shape, lambda i: (0,) * len(shape))


def _eblk(width):
    return pl.BlockSpec((EDGE_BLK, width), lambda i: (i, 0))


def _idxblk(width):
    return pl.BlockSpec((1, 1, width), lambda i: (i, 0, 0))


def _layer0_call(rowm, colm, eaT, emb, encW, encb, eW1, eb1, eW2, eb2,
                 nW1, nb1, nW2, nb2, zW, zb):
    nE = eaT.shape[1]
    grid = (nE // EDGE_BLK,)
    return pl.pallas_call(
        _layer0_body,
        grid=grid,
        in_specs=[
            _idxblk(EDGE_BLK), _idxblk(EDGE_BLK),
            pl.BlockSpec((ED, EDGE_BLK), lambda i: (0, i)),
            _full((VOCAB, H)), _full((ED, H)), _full((1, H)),
            _full((3 * H, H)), _full((1, H)), _full((H, H)), _full((1, H)),
            _full((2 * H, H)), _full((1, H)), _full((H, H)), _full((1, H)),
            _full((H, H)), _full((1, H)),
        ],
        out_specs=[_eblk(H), _eblk(H)],
        out_shape=[jax.ShapeDtypeStruct((nE, H), jnp.bfloat16),
                   jax.ShapeDtypeStruct((nE, H), jnp.float32)],
    )(rowm, colm, eaT, emb, encW, encb, eW1, eb1, eW2, eb2, nW1, nb1, nW2, nb2,
      zW, zb)


def _layer1_call(yr, z, nW2, nb2):
    nE = yr.shape[0]
    grid = (nE // EDGE_BLK,)
    return pl.pallas_call(
        _layer1_body,
        grid=grid,
        in_specs=[
            _eblk(H), _eblk(H),
            _full((H, H)), _full((1, H)),
        ],
        out_specs=_eblk(H),
        out_shape=jax.ShapeDtypeStruct((nE, H), jnp.float32),
    )(yr, z, nW2, nb2)


def _head_call(ga, gb, gc, gd, rW1, rb1, rW2, rb2):
    return pl.pallas_call(
        _head_body,
        in_specs=[
            pl.BlockSpec((G, H), lambda: (0, 0)),
            pl.BlockSpec((G, H), lambda: (0, 0)),
            pl.BlockSpec((G, H), lambda: (0, 0)),
            pl.BlockSpec((G, H), lambda: (0, 0)),
            pl.BlockSpec((H, H), lambda: (0, 0)),
            pl.BlockSpec((1, H), lambda: (0, 0)),
            pl.BlockSpec((H, 1), lambda: (0, 0)),
            pl.BlockSpec((1, 1), lambda: (0, 0)),
        ],
        out_specs=pl.BlockSpec((G, 1), lambda: (0, 0)),
        out_shape=jax.ShapeDtypeStruct((G, 1), jnp.float32),
    )(ga, gb, gc, gd, rW1, rb1, rW2, rb2)


def kernel(edge_index, edge_attr, batch, n_items, n_locs, emb, enc_W, enc_b,
           edge_W1, edge_b1, edge_W2, edge_b2,
           node_W1, node_b1, node_W2, node_b2,
           reg_W1, reg_b1, reg_W2, reg_b2):
    row = edge_index[0]
    col = edge_index[1]
    r2 = lambda b: b.reshape(1, H)
    E2 = E // 2

    rowm = (row % VOCAB).astype(jnp.int32)
    colm = (col % VOCAB).astype(jnp.int32)
    eaT = edge_attr.T
    coli = col.astype(jnp.int32)
    rowi = row.astype(jnp.int32)
    zrows = jnp.zeros((NPS, H), jnp.float32)
    bi = batch.astype(jnp.int32)

    def idx3(a):
        return a.reshape(a.shape[0] // EDGE_BLK, 1, EDGE_BLK)

    def sc3(a):
        return a.reshape(NW, a.shape[0] // (NW * K), K)

    l0_args = (emb, enc_W, r2(enc_b),
               edge_W1[0], r2(edge_b1[0]), edge_W2[0], r2(edge_b2[0]),
               node_W1[0], r2(node_b1[0]), node_W2[0], r2(node_b2[0]),
               node_W1[1][H:2 * H], r2(node_b1[1]))

    za, msg0a = _layer0_call(idx3(rowm[:E2]), idx3(colm[:E2]), eaT[:, :E2],
                             *l0_args)
    pa = _scatter_call(msg0a, sc3(coli[:E2]), zrows)
    zb, msg0b = _layer0_call(idx3(rowm[E2:]), idx3(colm[E2:]), eaT[:, E2:],
                             *l0_args)
    pb = _scatter_call(msg0b, sc3(coli[E2:]), zrows)

    y = _combine_call(pa, pb, node_W1[1][0:H])

    yra = _gather_call(y, sc3(rowi[:E2]))
    msg1a = _layer1_call(yra, za, node_W2[1], r2(node_b2[1]))
    yrb = _gather_call(y, sc3(rowi[E2:]))
    ga = _pool_scatter_call(msg1a, sc3(coli[:E2]), bi, zrows)
    msg1b = _layer1_call(yrb, zb, node_W2[1], r2(node_b2[1]))
    gb = _pool_scatter_call(msg1b, sc3(coli[E2:]), bi, zrows)

    out = _head_call(ga[0], ga[1], gb[0], gb[1], reg_W1, r2(reg_b1), reg_W2,
                     reg_b2.reshape(1, 1))
    return out.squeeze(-1)


# speedup vs baseline: 1.0035x; 1.0035x over previous
import functools

import jax
import jax.numpy as jnp
from jax import lax
from jax.experimental import pallas as pl
from jax.experimental.pallas import tpu as pltpu
from jax.experimental.pallas import tpu_sc as plsc

H = 128
N = 10240
E = 163840
ED = 16
G = 40
VOCAB = 256

EDGE_BLK = 2048
N_BLK = 2048


def _layer0_body(rowm_ref, colm_ref, eaT_ref, emb_ref, encW_ref, encb_ref,
                 eW1_ref, eb1_ref, eW2_ref, eb2_ref,
                 nW1_ref, nb1_ref, nW2_ref, nb2_ref,
                 zW_ref, zb_ref,
                 z_ref, msg_ref):
    rowm = rowm_ref[0, 0, :]
    colm = colm_ref[0, 0, :]
    ids = lax.broadcasted_iota(jnp.int32, (EDGE_BLK, VOCAB), 1)
    oh_r = (ids == rowm[:, None]).astype(jnp.float32)
    oh_c = (ids == colm[:, None]).astype(jnp.float32)
    emb = emb_ref[...]
    xr = jnp.dot(oh_r, emb, preferred_element_type=jnp.float32)
    xc = jnp.dot(oh_c, emb, preferred_element_type=jnp.float32)
    e0 = lax.dot_general(eaT_ref[...], encW_ref[...], (((0,), (0,)), ((), ())),
                         preferred_element_type=jnp.float32) + encb_ref[...]
    W1 = eW1_ref[...]
    h = (jnp.dot(xr, W1[0:H], preferred_element_type=jnp.float32)
         + jnp.dot(xc, W1[H:2 * H], preferred_element_type=jnp.float32)
         + jnp.dot(e0, W1[2 * H:3 * H], preferred_element_type=jnp.float32)
         + eb1_ref[...])
    h = jnp.maximum(h, 0.0)
    e1 = jnp.dot(h, eW2_ref[...], preferred_element_type=jnp.float32) + eb2_ref[...]
    nW1 = nW1_ref[...]
    m = (jnp.dot(xr, nW1[0:H], preferred_element_type=jnp.float32)
         + jnp.dot(e1, nW1[H:2 * H], preferred_element_type=jnp.float32)
         + nb1_ref[...])
    m = jnp.maximum(m, 0.0)
    msg = jnp.dot(m, nW2_ref[...], preferred_element_type=jnp.float32) + nb2_ref[...]
    z = jnp.dot(e1, zW_ref[...], preferred_element_type=jnp.float32) + zb_ref[...]
    z_ref[...] = z.astype(jnp.bfloat16)
    msg_ref[...] = msg


def _layer1_body(yr_ref, z_ref, nW2_ref, nb2_ref, msg_ref):
    m = jnp.maximum(yr_ref[...] + z_ref[...].astype(jnp.float32), 0.0)
    msg_ref[...] = jnp.dot(m, nW2_ref[...], preferred_element_type=jnp.float32) + nb2_ref[...]


def _head_body(ga_ref, gb_ref, gc_ref, gd_ref,
               rW1_ref, rb1_ref, rW2_ref, rb2_ref, out_ref):
    g = ga_ref[...] + gb_ref[...] + gc_ref[...] + gd_ref[...]
    hmid = jnp.maximum(jnp.dot(g, rW1_ref[...], preferred_element_type=jnp.float32)
                       + rb1_ref[...], 0.0)
    out_ref[...] = jnp.dot(hmid, rW2_ref[...], preferred_element_type=jnp.float32) + rb2_ref[...]


NW = 32
EPW = E // NW
K = 128
CHUNKS = EPW // K
S_BLKR = 128
S_SUB = S_BLKR // K
S_GROUPS = EPW // S_BLKR
G_BLKR = 256
G_SUB = G_BLKR // K
G_GROUPS = EPW // G_BLKR
NPS = N // 16


def _sc_mesh():
    return plsc.VectorSubcoreMesh(core_axis_name="c", subcore_axis_name="s")


def _scatter_call(msg, col3, zrows):
    epw = msg.shape[0] // NW
    sgroups = epw // S_BLKR

    @functools.partial(
        pl.kernel,
        out_type=jax.ShapeDtypeStruct((2, N, H), jnp.float32),
        mesh=_sc_mesh(),
        scratch_types=[
            pltpu.VMEM((2, S_SUB, K), jnp.int32),
            pltpu.VMEM((2, S_BLKR, H), jnp.float32),
            pltpu.VMEM_SHARED((N, H), jnp.float32),
            pltpu.SemaphoreType.DMA,
            pltpu.SemaphoreType.DMA,
            pltpu.SemaphoreType.DMA,
        ],
    )
    def k(msg_hbm, col_hbm, z_hbm, out_hbm, idx_v, rows_v, shared, sf0, sf1, ss):
        cid = lax.axis_index("c")
        sid = lax.axis_index("s")
        wid = sid * 2 + cid
        base_e = wid * epw
        sf = (sf0, sf1)

        pltpu.async_copy(col_hbm.at[wid, pl.ds(0, S_SUB)], idx_v.at[0], sf0)
        pltpu.async_copy(msg_hbm.at[pl.ds(base_e, S_BLKR)], rows_v.at[0], sf0)
        pltpu.sync_copy(z_hbm, shared.at[pl.ds(sid * NPS, NPS)])
        plsc.subcore_barrier()

        @pl.loop(0, sgroups, step=2)
        def _(g):
            for b in range(2):
                gg = g + b
                nb = 1 - b
                pltpu.make_async_copy(col_hbm.at[wid, pl.ds(0, S_SUB)],
                                      idx_v.at[b], sf[b]).wait()
                pltpu.make_async_copy(msg_hbm.at[pl.ds(base_e, S_BLKR)],
                                      rows_v.at[b], sf[b]).wait()

                @pl.when(gg + 1 < sgroups)
                def _():
                    pltpu.async_copy(col_hbm.at[wid, pl.ds((gg + 1) * S_SUB, S_SUB)],
                                     idx_v.at[nb], sf[nb])
                    pltpu.async_copy(
                        msg_hbm.at[pl.ds(base_e + (gg + 1) * S_BLKR, S_BLKR)],
                        rows_v.at[nb], sf[nb])

                for j in range(S_SUB):
                    pltpu.async_copy(rows_v.at[b, pl.ds(j * K, K)],
                                     shared.at[idx_v.at[b, j]], ss, add=True)
                for j in range(S_SUB):
                    pltpu.make_async_copy(rows_v.at[b, pl.ds(j * K, K)],
                                          shared.at[pl.ds(0, K)], ss).wait()

        plsc.subcore_barrier()
        pltpu.sync_copy(shared.at[pl.ds(sid * NPS, NPS)],
                        out_hbm.at[cid, pl.ds(sid * NPS, NPS)])

    return k(msg, col3, zrows)


def _pool_scatter_call(msg, col3, batch, zrows):
    epw = msg.shape[0] // NW
    sgroups = epw // S_BLKR

    @functools.partial(
        pl.kernel,
        out_type=jax.ShapeDtypeStruct((2, G, H), jnp.float32),
        mesh=_sc_mesh(),
        scratch_types=[
            pltpu.VMEM((2, S_SUB, K), jnp.int32),
            pltpu.VMEM((2, S_SUB, K), jnp.int32),
            pltpu.VMEM((2, S_BLKR, H), jnp.float32),
            pltpu.VMEM_SHARED((G, H), jnp.float32),
            pltpu.SemaphoreType.DMA,
            pltpu.SemaphoreType.DMA,
            pltpu.SemaphoreType.DMA,
            pltpu.SemaphoreType.DMA,
            pltpu.SemaphoreType.DMA,
        ],
    )
    def k(msg_hbm, col_hbm, b_hbm, z_hbm, out_hbm,
          idx_v, bc_v, rows_v, gacc, sf0, sf1, sb0, sb1, ss):
        cid = lax.axis_index("c")
        sid = lax.axis_index("s")
        wid = sid * 2 + cid
        base_e = wid * epw
        sf = (sf0, sf1)
        sb = (sb0, sb1)

        pltpu.async_copy(col_hbm.at[wid, pl.ds(0, S_SUB)], idx_v.at[0], sf0)
        pltpu.async_copy(msg_hbm.at[pl.ds(base_e, S_BLKR)], rows_v.at[0], sf0)

        @pl.when(sid == 0)
        def _():
            pltpu.sync_copy(z_hbm.at[pl.ds(0, G)], gacc)

        plsc.subcore_barrier()

        @pl.loop(0, sgroups, step=2)
        def _(g):
            for b in range(2):
                gg = g + b
                nb = 1 - b
                pltpu.make_async_copy(col_hbm.at[wid, pl.ds(0, S_SUB)],
                                      idx_v.at[b], sf[b]).wait()
                pltpu.make_async_copy(msg_hbm.at[pl.ds(base_e, S_BLKR)],
                                      rows_v.at[b], sf[b]).wait()

                for j in range(S_SUB):
                    pltpu.async_copy(b_hbm.at[idx_v.at[b, j]], bc_v.at[b, j], sb[b])

                @pl.when(gg + 1 < sgroups)
                def _():
                    pltpu.async_copy(col_hbm.at[wid, pl.ds((gg + 1) * S_SUB, S_SUB)],
                                     idx_v.at[nb], sf[nb])
                    pltpu.async_copy(
                        msg_hbm.at[pl.ds(base_e + (gg + 1) * S_BLKR, S_BLKR)],
                        rows_v.at[nb], sf[nb])

                for j in range(S_SUB):
                    pltpu.make_async_copy(b_hbm.at[pl.ds(0, K)],
                                          bc_v.at[b, j], sb[b]).wait()
                for j in range(S_SUB):
                    pltpu.async_copy(rows_v.at[b, pl.ds(j * K, K)],
                                     gacc.at[bc_v.at[b, j]], ss, add=True)
                for j in range(S_SUB):
                    pltpu.make_async_copy(msg_hbm.at[pl.ds(0, K)],
                                          rows_v.at[b, pl.ds(j * K, K)], ss).wait()

        plsc.subcore_barrier()

        @pl.when(sid == 0)
        def _():
            pltpu.sync_copy(gacc, out_hbm.at[cid])

    return k(msg, col3, batch, zrows)


def _gather_call(x1, row3):
    nE = row3.shape[0] * row3.shape[1] * row3.shape[2]
    epw = nE // NW
    ggroups = epw // G_BLKR

    @functools.partial(
        pl.kernel,
        out_type=jax.ShapeDtypeStruct((nE, H), jnp.float32),
        mesh=_sc_mesh(),
        scratch_types=[
            pltpu.VMEM((2, G_SUB, K), jnp.int32),
            pltpu.VMEM((2, G_BLKR, H), jnp.float32),
            pltpu.SemaphoreType.DMA,
            pltpu.SemaphoreType.DMA,
            pltpu.SemaphoreType.DMA,
            pltpu.SemaphoreType.DMA,
            pltpu.SemaphoreType.DMA,
        ],
    )
    def k(x_hbm, row_hbm, out_hbm, idx_v, rows_v, si0, si1, sg, so0, so1):
        cid = lax.axis_index("c")
        sid = lax.axis_index("s")
        wid = sid * 2 + cid
        base_e = wid * epw
        si = (si0, si1)
        so = (so0, so1)

        pltpu.async_copy(row_hbm.at[wid, pl.ds(0, G_SUB)], idx_v.at[0], si0)

        @pl.loop(0, ggroups, step=2)
        def _(g):
            for b in range(2):
                gg = g + b
                nb = 1 - b
                pltpu.make_async_copy(row_hbm.at[wid, pl.ds(0, G_SUB)],
                                      idx_v.at[b], si[b]).wait()

                @pl.when(gg + 1 < ggroups)
                def _():
                    pltpu.async_copy(row_hbm.at[wid, pl.ds((gg + 1) * G_SUB, G_SUB)],
                                     idx_v.at[nb], si[nb])

                @pl.when(gg >= 2)
                def _():
                    pltpu.make_async_copy(rows_v.at[b],
                                          out_hbm.at[pl.ds(0, G_BLKR)], so[b]).wait()

                for j in range(G_SUB):
                    pltpu.async_copy(x_hbm.at[idx_v.at[b, j]],
                                     rows_v.at[b, pl.ds(j * K, K)], sg)
                for j in range(G_SUB):
                    pltpu.make_async_copy(x_hbm.at[pl.ds(0, K)],
                                          rows_v.at[b, pl.ds(j * K, K)], sg).wait()
                pltpu.async_copy(rows_v.at[b],
                                 out_hbm.at[pl.ds(base_e + gg * G_BLKR, G_BLKR)], so[b])

        for b in range(2):
            pltpu.make_async_copy(rows_v.at[b],
                                  out_hbm.at[pl.ds(0, G_BLKR)], so[b]).wait()

    return k(x1, row3)


def _combine_body(a_ref, b_ref, c_ref, d_ref, dW_ref, y_ref):
    y_ref[...] = jnp.dot(a_ref[...] + b_ref[...] + c_ref[...] + d_ref[...],
                         dW_ref[...], preferred_element_type=jnp.float32)


def _combine_call(pa, pb, dW):
    grid = (N // N_BLK,)
    nblk = pl.BlockSpec((N_BLK, H), lambda i: (i, 0))
    return pl.pallas_call(
        _combine_body,
        grid=grid,
        in_specs=[nblk, nblk, nblk, nblk, _full((H, H))],
        out_specs=nblk,
        out_shape=jax.ShapeDtypeStruct((N, H), jnp.float32),
    )(pa[0], pa[1], pb[0], pb[1], dW)


def _full(shape):
    return pl.BlockSpec(shape, lambda i: (0,) * len(shape))


def _eblk(width):
    return pl.BlockSpec((EDGE_BLK, width), lambda i: (i, 0))


def _idxblk(width):
    return pl.BlockSpec((1, 1, width), lambda i: (i, 0, 0))


def _layer0_call(rowm, colm, eaT, emb, encW, encb, eW1, eb1, eW2, eb2,
                 nW1, nb1, nW2, nb2, zW, zb):
    nE = eaT.shape[1]
    grid = (nE // EDGE_BLK,)
    return pl.pallas_call(
        _layer0_body,
        grid=grid,
        in_specs=[
            _idxblk(EDGE_BLK), _idxblk(EDGE_BLK),
            pl.BlockSpec((ED, EDGE_BLK), lambda i: (0, i)),
            _full((VOCAB, H)), _full((ED, H)), _full((1, H)),
            _full((3 * H, H)), _full((1, H)), _full((H, H)), _full((1, H)),
            _full((2 * H, H)), _full((1, H)), _full((H, H)), _full((1, H)),
            _full((H, H)), _full((1, H)),
        ],
        out_specs=[_eblk(H), _eblk(H)],
        out_shape=[jax.ShapeDtypeStruct((nE, H), jnp.bfloat16),
                   jax.ShapeDtypeStruct((nE, H), jnp.float32)],
    )(rowm, colm, eaT, emb, encW, encb, eW1, eb1, eW2, eb2, nW1, nb1, nW2, nb2,
      zW, zb)


def _layer1_call(yr, z, nW2, nb2):
    nE = yr.shape[0]
    grid = (nE // EDGE_BLK,)
    return pl.pallas_call(
        _layer1_body,
        grid=grid,
        in_specs=[
            _eblk(H), _eblk(H),
            _full((H, H)), _full((1, H)),
        ],
        out_specs=_eblk(H),
        out_shape=jax.ShapeDtypeStruct((nE, H), jnp.float32),
    )(yr, z, nW2, nb2)


def _head_call(ga, gb, gc, gd, rW1, rb1, rW2, rb2):
    return pl.pallas_call(
        _head_body,
        in_specs=[
            pl.BlockSpec((G, H), lambda: (0, 0)),
            pl.BlockSpec((G, H), lambda: (0, 0)),
            pl.BlockSpec((G, H), lambda: (0, 0)),
            pl.BlockSpec((G, H), lambda: (0, 0)),
            pl.BlockSpec((H, H), lambda: (0, 0)),
            pl.BlockSpec((1, H), lambda: (0, 0)),
            pl.BlockSpec((H, 1), lambda: (0, 0)),
            pl.BlockSpec((1, 1), lambda: (0, 0)),
        ],
        out_specs=pl.BlockSpec((G, 1), lambda: (0, 0)),
        out_shape=jax.ShapeDtypeStruct((G, 1), jnp.float32),
    )(ga, gb, gc, gd, rW1, rb1, rW2, rb2)


def kernel(edge_index, edge_attr, batch, n_items, n_locs, emb, enc_W, enc_b,
           edge_W1, edge_b1, edge_W2, edge_b2,
           node_W1, node_b1, node_W2, node_b2,
           reg_W1, reg_b1, reg_W2, reg_b2):
    row = edge_index[0]
    col = edge_index[1]
    r2 = lambda b: b.reshape(1, H)
    E2 = E // 2

    rowm = (row % VOCAB).astype(jnp.int32)
    colm = (col % VOCAB).astype(jnp.int32)
    eaT = edge_attr.T
    coli = col.astype(jnp.int32)
    rowi = row.astype(jnp.int32)
    zrows = jnp.zeros((NPS, H), jnp.float32)
    bi = batch.astype(jnp.int32)

    def idx3(a):
        return a.reshape(a.shape[0] // EDGE_BLK, 1, EDGE_BLK)

    def sc3(a):
        return a.reshape(NW, a.shape[0] // (NW * K), K)

    l0_args = (emb, enc_W, r2(enc_b),
               edge_W1[0], r2(edge_b1[0]), edge_W2[0], r2(edge_b2[0]),
               node_W1[0], r2(node_b1[0]), node_W2[0], r2(node_b2[0]),
               node_W1[1][H:2 * H], r2(node_b1[1]))

    za, msg0a = _layer0_call(idx3(rowm[:E2]), idx3(colm[:E2]), eaT[:, :E2],
                             *l0_args)
    pa = _scatter_call(msg0a, sc3(coli[:E2]), zrows)
    zb, msg0b = _layer0_call(idx3(rowm[E2:]), idx3(colm[E2:]), eaT[:, E2:],
                             *l0_args)
    pb = _scatter_call(msg0b, sc3(coli[E2:]), zrows)

    y = _combine_call(pa, pb, node_W1[1][0:H])

    yra = _gather_call(y, sc3(rowi[:E2]))
    msg1a = _layer1_call(yra, za, node_W2[1], r2(node_b2[1]))
    yrb = _gather_call(y, sc3(rowi[E2:]))
    ga = _pool_scatter_call(msg1a, sc3(coli[:E2]), bi, zrows)
    msg1b = _layer1_call(yrb, zb, node_W2[1], r2(node_b2[1]))
    gb = _pool_scatter_call(msg1b, sc3(coli[E2:]), bi, zrows)

    out = _head_call(ga[0], ga[1], gb[0], gb[1], reg_W1, r2(reg_b1), reg_W2,
                     reg_b2.reshape(1, 1))
    return out.squeeze(-1)


# speedup vs baseline: 1.0037x; 1.0002x over previous
import functools

import jax
import jax.numpy as jnp
from jax import lax
from jax.experimental import pallas as pl
from jax.experimental.pallas import tpu as pltpu
from jax.experimental.pallas import tpu_sc as plsc

H = 128
N = 10240
E = 163840
ED = 16
G = 40
VOCAB = 256

EDGE_BLK = 2048
N_BLK = 2048


def _layer0_body(rowm_ref, colm_ref, eaT_ref, emb_ref, encW_ref, encb_ref,
                 eW1_ref, eb1_ref, eW2_ref, eb2_ref,
                 nW1_ref, nb1_ref, nW2_ref, nb2_ref,
                 zW_ref, zb_ref,
                 z_ref, msg_ref):
    rowm = lax.rem(rowm_ref[0, 0, :], VOCAB)
    colm = lax.rem(colm_ref[0, 0, :], VOCAB)
    ids = lax.broadcasted_iota(jnp.int32, (EDGE_BLK, VOCAB), 1)
    oh_r = (ids == rowm[:, None]).astype(jnp.float32)
    oh_c = (ids == colm[:, None]).astype(jnp.float32)
    emb = emb_ref[...]
    xr = jnp.dot(oh_r, emb, preferred_element_type=jnp.float32)
    xc = jnp.dot(oh_c, emb, preferred_element_type=jnp.float32)
    e0 = lax.dot_general(eaT_ref[...], encW_ref[...], (((0,), (0,)), ((), ())),
                         preferred_element_type=jnp.float32) + encb_ref[...]
    W1 = eW1_ref[...]
    h = (jnp.dot(xr, W1[0:H], preferred_element_type=jnp.float32)
         + jnp.dot(xc, W1[H:2 * H], preferred_element_type=jnp.float32)
         + jnp.dot(e0, W1[2 * H:3 * H], preferred_element_type=jnp.float32)
         + eb1_ref[...])
    h = jnp.maximum(h, 0.0)
    e1 = jnp.dot(h, eW2_ref[...], preferred_element_type=jnp.float32) + eb2_ref[...]
    nW1 = nW1_ref[...]
    m = (jnp.dot(xr, nW1[0:H], preferred_element_type=jnp.float32)
         + jnp.dot(e1, nW1[H:2 * H], preferred_element_type=jnp.float32)
         + nb1_ref[...])
    m = jnp.maximum(m, 0.0)
    msg = jnp.dot(m, nW2_ref[...], preferred_element_type=jnp.float32) + nb2_ref[...]
    z = jnp.dot(e1, zW_ref[...], preferred_element_type=jnp.float32) + zb_ref[...]
    z_ref[...] = z.astype(jnp.bfloat16)
    msg_ref[...] = msg


def _layer1_body(yr_ref, z_ref, nW2_ref, nb2_ref, msg_ref):
    m = jnp.maximum(yr_ref[...].astype(jnp.float32) + z_ref[...].astype(jnp.float32), 0.0)
    msg_ref[...] = jnp.dot(m, nW2_ref[...], preferred_element_type=jnp.float32) + nb2_ref[...]


def _head_body(ga_ref, gb_ref, gc_ref, gd_ref,
               rW1_ref, rb1_ref, rW2_ref, rb2_ref, out_ref):
    g = ga_ref[...] + gb_ref[...] + gc_ref[...] + gd_ref[...]
    hmid = jnp.maximum(jnp.dot(g, rW1_ref[...], preferred_element_type=jnp.float32)
                       + rb1_ref[...], 0.0)
    out_ref[...] = jnp.dot(hmid, rW2_ref[...], preferred_element_type=jnp.float32) + rb2_ref[...]


NW = 32
EPW = E // NW
K = 128
CHUNKS = EPW // K
S_BLKR = 128
S_SUB = S_BLKR // K
S_GROUPS = EPW // S_BLKR
G_BLKR = 256
G_SUB = G_BLKR // K
G_GROUPS = EPW // G_BLKR
NPS = N // 16


def _sc_mesh():
    return plsc.VectorSubcoreMesh(core_axis_name="c", subcore_axis_name="s")


def _scatter_call(msg, col3, zrows):
    epw = msg.shape[0] // NW
    sgroups = epw // S_BLKR

    @functools.partial(
        pl.kernel,
        out_type=jax.ShapeDtypeStruct((2, N, H), jnp.float32),
        mesh=_sc_mesh(),
        scratch_types=[
            pltpu.VMEM((2, S_SUB, K), jnp.int32),
            pltpu.VMEM((2, S_BLKR, H), jnp.float32),
            pltpu.VMEM_SHARED((N, H), jnp.float32),
            pltpu.SemaphoreType.DMA,
            pltpu.SemaphoreType.DMA,
            pltpu.SemaphoreType.DMA,
        ],
    )
    def k(msg_hbm, col_hbm, z_hbm, out_hbm, idx_v, rows_v, shared, sf0, sf1, ss):
        cid = lax.axis_index("c")
        sid = lax.axis_index("s")
        wid = sid * 2 + cid
        base_e = wid * epw
        sf = (sf0, sf1)

        pltpu.async_copy(col_hbm.at[wid, pl.ds(0, S_SUB)], idx_v.at[0], sf0)
        pltpu.async_copy(msg_hbm.at[pl.ds(base_e, S_BLKR)], rows_v.at[0], sf0)
        pltpu.sync_copy(z_hbm, shared.at[pl.ds(sid * NPS, NPS)])
        plsc.subcore_barrier()

        @pl.loop(0, sgroups, step=2)
        def _(g):
            for b in range(2):
                gg = g + b
                nb = 1 - b
                pltpu.make_async_copy(col_hbm.at[wid, pl.ds(0, S_SUB)],
                                      idx_v.at[b], sf[b]).wait()
                pltpu.make_async_copy(msg_hbm.at[pl.ds(base_e, S_BLKR)],
                                      rows_v.at[b], sf[b]).wait()

                @pl.when(gg + 1 < sgroups)
                def _():
                    pltpu.async_copy(col_hbm.at[wid, pl.ds((gg + 1) * S_SUB, S_SUB)],
                                     idx_v.at[nb], sf[nb])
                    pltpu.async_copy(
                        msg_hbm.at[pl.ds(base_e + (gg + 1) * S_BLKR, S_BLKR)],
                        rows_v.at[nb], sf[nb])

                for j in range(S_SUB):
                    pltpu.async_copy(rows_v.at[b, pl.ds(j * K, K)],
                                     shared.at[idx_v.at[b, j]], ss, add=True)
                for j in range(S_SUB):
                    pltpu.make_async_copy(rows_v.at[b, pl.ds(j * K, K)],
                                          shared.at[pl.ds(0, K)], ss).wait()

        plsc.subcore_barrier()
        pltpu.sync_copy(shared.at[pl.ds(sid * NPS, NPS)],
                        out_hbm.at[cid, pl.ds(sid * NPS, NPS)])

    return k(msg, col3, zrows)


def _pool_scatter_call(msg, col3, batch, zrows):
    epw = msg.shape[0] // NW
    sgroups = epw // S_BLKR

    @functools.partial(
        pl.kernel,
        out_type=jax.ShapeDtypeStruct((2, G, H), jnp.float32),
        mesh=_sc_mesh(),
        scratch_types=[
            pltpu.VMEM((2, S_SUB, K), jnp.int32),
            pltpu.VMEM((2, S_SUB, K), jnp.int32),
            pltpu.VMEM((2, S_BLKR, H), jnp.float32),
            pltpu.VMEM_SHARED((G, H), jnp.float32),
            pltpu.SemaphoreType.DMA,
            pltpu.SemaphoreType.DMA,
            pltpu.SemaphoreType.DMA,
            pltpu.SemaphoreType.DMA,
            pltpu.SemaphoreType.DMA,
        ],
    )
    def k(msg_hbm, col_hbm, b_hbm, z_hbm, out_hbm,
          idx_v, bc_v, rows_v, gacc, sf0, sf1, sb0, sb1, ss):
        cid = lax.axis_index("c")
        sid = lax.axis_index("s")
        wid = sid * 2 + cid
        base_e = wid * epw
        sf = (sf0, sf1)
        sb = (sb0, sb1)

        pltpu.async_copy(col_hbm.at[wid, pl.ds(0, S_SUB)], idx_v.at[0], sf0)
        pltpu.async_copy(msg_hbm.at[pl.ds(base_e, S_BLKR)], rows_v.at[0], sf0)

        @pl.when(sid == 0)
        def _():
            pltpu.sync_copy(z_hbm.at[pl.ds(0, G)], gacc)

        plsc.subcore_barrier()

        @pl.loop(0, sgroups, step=2)
        def _(g):
            for b in range(2):
                gg = g + b
                nb = 1 - b
                pltpu.make_async_copy(col_hbm.at[wid, pl.ds(0, S_SUB)],
                                      idx_v.at[b], sf[b]).wait()
                pltpu.make_async_copy(msg_hbm.at[pl.ds(base_e, S_BLKR)],
                                      rows_v.at[b], sf[b]).wait()

                for j in range(S_SUB):
                    pltpu.async_copy(b_hbm.at[idx_v.at[b, j]], bc_v.at[b, j], sb[b])

                @pl.when(gg + 1 < sgroups)
                def _():
                    pltpu.async_copy(col_hbm.at[wid, pl.ds((gg + 1) * S_SUB, S_SUB)],
                                     idx_v.at[nb], sf[nb])
                    pltpu.async_copy(
                        msg_hbm.at[pl.ds(base_e + (gg + 1) * S_BLKR, S_BLKR)],
                        rows_v.at[nb], sf[nb])

                for j in range(S_SUB):
                    pltpu.make_async_copy(b_hbm.at[pl.ds(0, K)],
                                          bc_v.at[b, j], sb[b]).wait()
                for j in range(S_SUB):
                    pltpu.async_copy(rows_v.at[b, pl.ds(j * K, K)],
                                     gacc.at[bc_v.at[b, j]], ss, add=True)
                for j in range(S_SUB):
                    pltpu.make_async_copy(msg_hbm.at[pl.ds(0, K)],
                                          rows_v.at[b, pl.ds(j * K, K)], ss).wait()

        plsc.subcore_barrier()

        @pl.when(sid == 0)
        def _():
            pltpu.sync_copy(gacc, out_hbm.at[cid])

    return k(msg, col3, batch, zrows)


def _gather_call(x1, row3):
    nE = row3.shape[0] * row3.shape[1] * row3.shape[2]
    epw = nE // NW
    ggroups = epw // G_BLKR

    @functools.partial(
        pl.kernel,
        out_type=jax.ShapeDtypeStruct((nE, H), jnp.float32),
        mesh=_sc_mesh(),
        scratch_types=[
            pltpu.VMEM((2, G_SUB, K), jnp.int32),
            pltpu.VMEM((2, G_BLKR, H), jnp.float32),
            pltpu.SemaphoreType.DMA,
            pltpu.SemaphoreType.DMA,
            pltpu.SemaphoreType.DMA,
            pltpu.SemaphoreType.DMA,
            pltpu.SemaphoreType.DMA,
        ],
    )
    def k(x_hbm, row_hbm, out_hbm, idx_v, rows_v, si0, si1, sg, so0, so1):
        cid = lax.axis_index("c")
        sid = lax.axis_index("s")
        wid = sid * 2 + cid
        base_e = wid * epw
        si = (si0, si1)
        so = (so0, so1)

        pltpu.async_copy(row_hbm.at[wid, pl.ds(0, G_SUB)], idx_v.at[0], si0)

        @pl.loop(0, ggroups, step=2)
        def _(g):
            for b in range(2):
                gg = g + b
                nb = 1 - b
                pltpu.make_async_copy(row_hbm.at[wid, pl.ds(0, G_SUB)],
                                      idx_v.at[b], si[b]).wait()

                @pl.when(gg + 1 < ggroups)
                def _():
                    pltpu.async_copy(row_hbm.at[wid, pl.ds((gg + 1) * G_SUB, G_SUB)],
                                     idx_v.at[nb], si[nb])

                @pl.when(gg >= 2)
                def _():
                    pltpu.make_async_copy(rows_v.at[b],
                                          out_hbm.at[pl.ds(0, G_BLKR)], so[b]).wait()

                for j in range(G_SUB):
                    pltpu.async_copy(x_hbm.at[idx_v.at[b, j]],
                                     rows_v.at[b, pl.ds(j * K, K)], sg)
                for j in range(G_SUB):
                    pltpu.make_async_copy(x_hbm.at[pl.ds(0, K)],
                                          rows_v.at[b, pl.ds(j * K, K)], sg).wait()
                pltpu.async_copy(rows_v.at[b],
                                 out_hbm.at[pl.ds(base_e + gg * G_BLKR, G_BLKR)], so[b])

        for b in range(2):
            pltpu.make_async_copy(rows_v.at[b],
                                  out_hbm.at[pl.ds(0, G_BLKR)], so[b]).wait()

    return k(x1, row3)


def _combine_body(a_ref, b_ref, c_ref, d_ref, dW_ref, y_ref):
    y_ref[...] = jnp.dot(a_ref[...] + b_ref[...] + c_ref[...] + d_ref[...],
                         dW_ref[...], preferred_element_type=jnp.float32)


def _combine_call(pa, pb, dW):
    grid = (N // N_BLK,)
    nblk = pl.BlockSpec((N_BLK, H), lambda i: (i, 0))
    return pl.pallas_call(
        _combine_body,
        grid=grid,
        in_specs=[nblk, nblk, nblk, nblk, _full((H, H))],
        out_specs=nblk,
        out_shape=jax.ShapeDtypeStruct((N, H), jnp.float32),
    )(pa[0], pa[1], pb[0], pb[1], dW)


def _full(shape):
    return pl.BlockSpec(shape, lambda i: (0,) * len(shape))


def _eblk(width):
    return pl.BlockSpec((EDGE_BLK, width), lambda i: (i, 0))


def _idxblk(width):
    return pl.BlockSpec((1, 1, width), lambda i: (i, 0, 0))


def _layer0_call(rowm, colm, eaT, emb, encW, encb, eW1, eb1, eW2, eb2,
                 nW1, nb1, nW2, nb2, zW, zb):
    nE = eaT.shape[1]
    grid = (nE // EDGE_BLK,)
    return pl.pallas_call(
        _layer0_body,
        grid=grid,
        in_specs=[
            _idxblk(EDGE_BLK), _idxblk(EDGE_BLK),
            pl.BlockSpec((ED, EDGE_BLK), lambda i: (0, i)),
            _full((VOCAB, H)), _full((ED, H)), _full((1, H)),
            _full((3 * H, H)), _full((1, H)), _full((H, H)), _full((1, H)),
            _full((2 * H, H)), _full((1, H)), _full((H, H)), _full((1, H)),
            _full((H, H)), _full((1, H)),
        ],
        out_specs=[_eblk(H), _eblk(H)],
        out_shape=[jax.ShapeDtypeStruct((nE, H), jnp.bfloat16),
                   jax.ShapeDtypeStruct((nE, H), jnp.float32)],
    )(rowm, colm, eaT, emb, encW, encb, eW1, eb1, eW2, eb2, nW1, nb1, nW2, nb2,
      zW, zb)


def _layer1_call(yr, z, nW2, nb2):
    nE = yr.shape[0]
    grid = (nE // EDGE_BLK,)
    return pl.pallas_call(
        _layer1_body,
        grid=grid,
        in_specs=[
            _eblk(H), _eblk(H),
            _full((H, H)), _full((1, H)),
        ],
        out_specs=_eblk(H),
        out_shape=jax.ShapeDtypeStruct((nE, H), jnp.float32),
    )(yr, z, nW2, nb2)


def _head_call(ga, gb, gc, gd, rW1, rb1, rW2, rb2):
    return pl.pallas_call(
        _head_body,
        in_specs=[
            pl.BlockSpec((G, H), lambda: (0, 0)),
            pl.BlockSpec((G, H), lambda: (0, 0)),
            pl.BlockSpec((G, H), lambda: (0, 0)),
            pl.BlockSpec((G, H), lambda: (0, 0)),
            pl.BlockSpec((H, H), lambda: (0, 0)),
            pl.BlockSpec((1, H), lambda: (0, 0)),
            pl.BlockSpec((H, 1), lambda: (0, 0)),
            pl.BlockSpec((1, 1), lambda: (0, 0)),
        ],
        out_specs=pl.BlockSpec((G, 1), lambda: (0, 0)),
        out_shape=jax.ShapeDtypeStruct((G, 1), jnp.float32),
    )(ga, gb, gc, gd, rW1, rb1, rW2, rb2)


def kernel(edge_index, edge_attr, batch, n_items, n_locs, emb, enc_W, enc_b,
           edge_W1, edge_b1, edge_W2, edge_b2,
           node_W1, node_b1, node_W2, node_b2,
           reg_W1, reg_b1, reg_W2, reg_b2):
    row = edge_index[0]
    col = edge_index[1]
    r2 = lambda b: b.reshape(1, H)
    E2 = E // 2

    eaT = edge_attr.T
    coli = col.astype(jnp.int32)
    rowi = row.astype(jnp.int32)
    zrows = jnp.zeros((NPS, H), jnp.float32)
    bi = batch.astype(jnp.int32)

    def idx3(a):
        return a.reshape(a.shape[0] // EDGE_BLK, 1, EDGE_BLK)

    def sc3(a):
        return a.reshape(NW, a.shape[0] // (NW * K), K)

    l0_args = (emb, enc_W, r2(enc_b),
               edge_W1[0], r2(edge_b1[0]), edge_W2[0], r2(edge_b2[0]),
               node_W1[0], r2(node_b1[0]), node_W2[0], r2(node_b2[0]),
               node_W1[1][H:2 * H], r2(node_b1[1]))

    za, msg0a = _layer0_call(idx3(rowi[:E2]), idx3(coli[:E2]), eaT[:, :E2],
                             *l0_args)
    pa = _scatter_call(msg0a, sc3(coli[:E2]), zrows)
    zb, msg0b = _layer0_call(idx3(rowi[E2:]), idx3(coli[E2:]), eaT[:, E2:],
                             *l0_args)
    pb = _scatter_call(msg0b, sc3(coli[E2:]), zrows)

    y = _combine_call(pa, pb, node_W1[1][0:H])

    yra = _gather_call(y, sc3(rowi[:E2]))
    msg1a = _layer1_call(yra, za, node_W2[1], r2(node_b2[1]))
    yrb = _gather_call(y, sc3(rowi[E2:]))
    ga = _pool_scatter_call(msg1a, sc3(coli[:E2]), bi, zrows)
    msg1b = _layer1_call(yrb, zb, node_W2[1], r2(node_b2[1]))
    gb = _pool_scatter_call(msg1b, sc3(coli[E2:]), bi, zrows)

    out = _head_call(ga[0], ga[1], gb[0], gb[1], reg_W1, r2(reg_b1), reg_W2,
                     reg_b2.reshape(1, 1))
    return out.squeeze(-1)
